# initial kernel scaffold (unmeasured)
import jax
import jax.numpy as jnp
from jax import lax
from jax.experimental import pallas as pl
from jax.experimental.pallas import tpu as pltpu


def kernel(
    x,
):
    def body(*refs):
        pass

    out_shape = jax.ShapeDtypeStruct(..., jnp.float32)
    return pl.pallas_call(body, out_shape=out_shape)(...)


# baseline (device time: 14330 ns/iter reference)
import jax
import jax.numpy as jnp
from jax import lax
from jax.experimental import pallas as pl
from jax.experimental.pallas import tpu as pltpu

N_DEV = 8
ROUNDS = 3


def kernel(x):
    m_per, n = x.shape

    def body(x_ref, out_ref, acc_ref, recv_ref, send_sems, recv_sems):
        my = lax.axis_index("i")

        xv = x_ref[:, :]
        row = lax.broadcasted_iota(jnp.int32, (m_per, n), 0)
        maxv = jnp.max(xv, axis=0)
        mask = xv == maxv[None, :]
        local_idx = jnp.min(jnp.where(mask, row, m_per), axis=0)
        acc_ref[0, :] = maxv
        acc_ref[1, :] = (my * m_per + local_idx).astype(jnp.float32)

        for r in range(ROUNDS):
            partner = my ^ (1 << r)
            rdma = pltpu.make_async_remote_copy(
                src_ref=acc_ref,
                dst_ref=recv_ref.at[r],
                send_sem=send_sems.at[r],
                recv_sem=recv_sems.at[r],
                device_id=(partner,),
                device_id_type=pl.DeviceIdType.MESH,
            )
            rdma.start()
            rdma.wait()

            mv, mi = acc_ref[0, :], acc_ref[1, :]
            ov, oi = recv_ref[r, 0, :], recv_ref[r, 1, :]
            take_mine = (mv > ov) | ((mv == ov) & (mi <= oi))
            acc_ref[0, :] = jnp.where(take_mine, mv, ov)
            acc_ref[1, :] = jnp.where(take_mine, mi, oi)

        out_ref[:, :] = acc_ref[:, :]

    return pl.pallas_call(
        body,
        out_shape=jax.ShapeDtypeStruct((2, n), jnp.float32),
        in_specs=[pl.BlockSpec(memory_space=pltpu.VMEM)],
        out_specs=pl.BlockSpec(memory_space=pltpu.VMEM),
        scratch_shapes=[
            pltpu.VMEM((2, n), jnp.float32),
            pltpu.VMEM((ROUNDS, 2, n), jnp.float32),
            pltpu.SemaphoreType.DMA((ROUNDS,)),
            pltpu.SemaphoreType.DMA((ROUNDS,)),
        ],
    )(x)


# device time: 7929 ns/iter; 1.8073x vs baseline; 1.8073x over previous
import jax
import jax.numpy as jnp
from jax import lax
from jax.experimental import pallas as pl
from jax.experimental.pallas import tpu as pltpu

N_DEV = 8


def kernel(x):
    m_per, n = x.shape

    def body(x_ref, out_ref, cand_ref, send_sems, recv_sems):
        my = lax.axis_index("i")

        barrier_sem = pltpu.get_barrier_semaphore()
        for p in range(1, N_DEV):
            pl.semaphore_signal(
                barrier_sem, inc=1,
                device_id=((my + p) % N_DEV,),
                device_id_type=pl.DeviceIdType.MESH,
            )

        xv = x_ref[:, :]
        row = lax.broadcasted_iota(jnp.int32, (m_per, n), 0)
        maxv = jnp.max(xv, axis=0)
        mask = xv == maxv[None, :]
        local_idx = jnp.min(jnp.where(mask, row, m_per), axis=0)
        cand_ref[my, 0, :] = maxv
        cand_ref[my, 1, :] = (my * m_per + local_idx).astype(jnp.float32)

        pl.semaphore_wait(barrier_sem, N_DEV - 1)

        rdmas = []
        for p in range(1, N_DEV):
            tgt = (my + p) % N_DEV
            rdma = pltpu.make_async_remote_copy(
                src_ref=cand_ref.at[my],
                dst_ref=cand_ref.at[my],
                send_sem=send_sems.at[p],
                recv_sem=recv_sems.at[my],
                device_id=(tgt,),
                device_id_type=pl.DeviceIdType.MESH,
            )
            rdma.start()
            rdmas.append(rdma)

        for p in range(1, N_DEV):
            src = (my + p) % N_DEV
            recv = pltpu.make_async_remote_copy(
                src_ref=cand_ref.at[src],
                dst_ref=cand_ref.at[src],
                send_sem=send_sems.at[p],
                recv_sem=recv_sems.at[src],
                device_id=(src,),
                device_id_type=pl.DeviceIdType.MESH,
            )
            recv.wait_recv()

        vals = cand_ref[:, 0, :]
        idxs = cand_ref[:, 1, :]
        gmax = jnp.max(vals, axis=0)
        tied = vals == gmax[None, :]
        gidx = jnp.min(jnp.where(tied, idxs, jnp.float32(2 * N_DEV * m_per)), axis=0)
        out_ref[0, :] = gmax
        out_ref[1, :] = gidx

        for rdma in rdmas:
            rdma.wait_send()

    return pl.pallas_call(
        body,
        out_shape=jax.ShapeDtypeStruct((2, n), jnp.float32),
        in_specs=[pl.BlockSpec(memory_space=pltpu.VMEM)],
        out_specs=pl.BlockSpec(memory_space=pltpu.VMEM),
        scratch_shapes=[
            pltpu.VMEM((N_DEV, 2, n), jnp.float32),
            pltpu.SemaphoreType.DMA((N_DEV,)),
            pltpu.SemaphoreType.DMA((N_DEV,)),
        ],
        compiler_params=pltpu.CompilerParams(collective_id=0),
    )(x)
